# Initial kernel scaffold; baseline (speedup 1.0000x reference)
import jax
import jax.numpy as jnp
from jax.experimental import pallas as pl


def kernel(hidden_states, norm_w, gate_w, w13, w2, shared_gate_w, shared_up_w, shared_down_w):
    raise NotImplementedError("write your pallas kernel here")


# speedup vs baseline: 1.5285x; 1.5285x over previous
import functools

import jax
import jax.numpy as jnp
from jax.experimental import pallas as pl

H = 768
I = 512
E = 8
K = 2
EPS = 1e-6
TB = 256


def _moe_block_kernel(x_ref, norm_w_ref, gate_w_ref, w13_ref, w2_ref,
                      shg_ref, shu_ref, shd_ref, out_ref):
    xb = x_ref[...]
    var = jnp.mean(jnp.square(xb), axis=-1, keepdims=True)
    xn = xb * jax.lax.rsqrt(var + EPS) * norm_w_ref[...]

    logits = jnp.dot(xn, gate_w_ref[...].T, preferred_element_type=jnp.float32)
    logits = logits - jnp.max(logits, axis=-1, keepdims=True)
    ex = jnp.exp(logits)
    probs = ex / jnp.sum(ex, axis=-1, keepdims=True)
    eidx = jax.lax.broadcasted_iota(jnp.int32, (TB, E), 1)
    i1 = jnp.argmax(probs, axis=-1)[:, None]
    one1 = (eidx == i1)
    p2 = jnp.where(one1, -jnp.inf, probs)
    i2 = jnp.argmax(p2, axis=-1)[:, None]
    one2 = (eidx == i2)
    v1 = jnp.max(probs, axis=-1, keepdims=True)
    v2 = jnp.max(p2, axis=-1, keepdims=True)
    denom = v1 + v2
    gates = (jnp.where(one1, v1, 0.0) + jnp.where(one2, v2, 0.0)) / denom

    xb16 = xn.astype(jnp.bfloat16)
    acc = jnp.zeros((TB, H), jnp.float32)
    for e in range(E):
        gu = jnp.dot(xb16, w13_ref[e], preferred_element_type=jnp.float32)
        g, u = gu[:, :I], gu[:, I:]
        act = (jax.nn.silu(g) * u).astype(jnp.bfloat16)
        oe = jnp.dot(act, w2_ref[e], preferred_element_type=jnp.float32)
        acc = acc + gates[:, e:e + 1] * oe

    sg = jnp.dot(xb16, shg_ref[...], preferred_element_type=jnp.float32)
    su = jnp.dot(xb16, shu_ref[...], preferred_element_type=jnp.float32)
    sh = (jax.nn.silu(sg) * su).astype(jnp.bfloat16)
    shared = jnp.dot(sh, shd_ref[...], preferred_element_type=jnp.float32)

    out_ref[...] = xb + acc + shared


@functools.partial(jax.jit, static_argnames=())
def kernel(hidden_states, norm_w, gate_w, w13, w2, shared_gate_w,
           shared_up_w, shared_down_w):
    B, S, _ = hidden_states.shape
    T = B * S
    x = hidden_states.reshape(T, H)
    w13_16 = w13.astype(jnp.bfloat16)
    w2_16 = w2.astype(jnp.bfloat16)
    shg16 = shared_gate_w.astype(jnp.bfloat16)
    shu16 = shared_up_w.astype(jnp.bfloat16)
    shd16 = shared_down_w.astype(jnp.bfloat16)

    grid = (T // TB,)
    out = pl.pallas_call(
        _moe_block_kernel,
        grid=grid,
        in_specs=[
            pl.BlockSpec((TB, H), lambda i: (i, 0)),
            pl.BlockSpec((H,), lambda i: (0,)),
            pl.BlockSpec((E, H), lambda i: (0, 0)),
            pl.BlockSpec((E, H, 2 * I), lambda i: (0, 0, 0)),
            pl.BlockSpec((E, I, H), lambda i: (0, 0, 0)),
            pl.BlockSpec((H, I), lambda i: (0, 0)),
            pl.BlockSpec((H, I), lambda i: (0, 0)),
            pl.BlockSpec((I, H), lambda i: (0, 0)),
        ],
        out_specs=pl.BlockSpec((TB, H), lambda i: (i, 0)),
        out_shape=jax.ShapeDtypeStruct((T, H), jnp.float32),
    )(x, norm_w, gate_w, w13_16, w2_16, shg16, shu16, shd16)
    return out.reshape(B, S, H)
